# Initial kernel scaffold; baseline (speedup 1.0000x reference)
import jax
import jax.numpy as jnp
from jax.experimental import pallas as pl


def kernel(u_pred, mag_pred, u_obs, mag_obs, thresh_s2_param):
    raise NotImplementedError("write your pallas kernel here")


# speedup vs baseline: 3.2070x; 3.2070x over previous
import functools
import math

import jax
import jax.numpy as jnp
import numpy as np
from jax import lax
from jax.experimental import pallas as pl
from jax.experimental.pallas import tpu as pltpu
from jax.experimental.pallas import tpu_sc as plsc

SPACE_DIMS = 3
N_SEG = 16
ROW = 2048
LANES = 16
NITER = ROW // LANES
NC = 2
NS = 16
SEG_PER_CORE = N_SEG // NC

def _deg2dist(deg):
    return 2.0 * np.sin(np.radians(np.asarray(deg, dtype=np.float64)) / 2.0)

_T_MIN = np.float32(_deg2dist(10.0 / 3600.0) ** 2)
_T_MAX = np.float32(_deg2dist(1.0) ** 2)
_LOG_RANGE = np.float32(np.log(np.float64(_T_MAX) / np.float64(_T_MIN)))
_SIGMA = np.float32(np.e)
_COEF = np.float32(np.float32(1.0 / np.sqrt(2.0 * np.pi)) / _SIGMA)
_LN2 = np.float32(0.693147180559945309)


def _logf(x):
    ix = plsc.bitcast(x, jnp.int32)
    ix = ix + (0x3F800000 - 0x3F3504F3)
    k = lax.shift_right_arithmetic(ix, 23) - 127
    mx = (ix & 0x007FFFFF) + 0x3F3504F3
    m = plsc.bitcast(mx, jnp.float32)
    f = m - 1.0
    s = f / (2.0 + f)
    z = s * s
    w = z * z
    t1 = w * (np.float32(0.40000972152) + w * np.float32(0.24279078841))
    t2 = z * (np.float32(0.66666662693) + w * np.float32(0.28498786688))
    r = t2 + t1
    hfsq = np.float32(0.5) * f * f
    return f - (hfsq - s * (hfsq + r)) + k.astype(jnp.float32) * _LN2


@functools.partial(
    pl.kernel,
    out_type=(
        jax.ShapeDtypeStruct((N_SEG, LANES), jnp.float32),
        jax.ShapeDtypeStruct((N_SEG, LANES), jnp.float32),
        jax.ShapeDtypeStruct((N_SEG, LANES), jnp.float32),
    ),
    mesh=plsc.VectorSubcoreMesh(
        core_axis_name="c", subcore_axis_name="s", num_cores=NC, num_subcores=NS
    ),
    compiler_params=pltpu.CompilerParams(needs_layout_passes=False),
    scratch_types=[
        pltpu.VMEM((ROW,), jnp.float32),
        pltpu.VMEM((ROW,), jnp.float32),
        pltpu.VMEM((ROW,), jnp.float32),
        pltpu.VMEM((ROW,), jnp.float32),
        pltpu.VMEM((ROW,), jnp.float32),
        pltpu.VMEM((ROW,), jnp.float32),
        pltpu.VMEM((ROW,), jnp.float32),
        pltpu.VMEM((ROW,), jnp.float32),
        pltpu.VMEM((LANES,), jnp.float32),
        pltpu.VMEM((ROW,), jnp.float32),
        pltpu.VMEM((ROW,), jnp.float32),
        pltpu.VMEM((LANES,), jnp.float32),
        pltpu.VMEM((LANES,), jnp.float32),
        pltpu.VMEM((LANES,), jnp.float32),
    ],
)
def _tscore(
    upx_h, upy_h, upz_h, uox_h, uoy_h, uoz_h, mp_h, mo_h, thp_h,
    score_h, hits_h, ll_h,
    upx, upy, upz, uox, uoy, uoz, mp, mo, thp,
    p_buf, c_buf, stage_p, stage_hh, stage_l,
):
    ci = lax.axis_index("c")
    si = lax.axis_index("s")
    active = si < SEG_PER_CORE
    seg = ci * SEG_PER_CORE + si
    lane = lax.iota(jnp.int32, LANES)

    @pl.when(active)
    def _work():
        base = seg * ROW
        pltpu.sync_copy(upx_h.at[pl.ds(base, ROW)], upx)
        pltpu.sync_copy(upy_h.at[pl.ds(base, ROW)], upy)
        pltpu.sync_copy(upz_h.at[pl.ds(base, ROW)], upz)
        pltpu.sync_copy(uox_h.at[pl.ds(base, ROW)], uox)
        pltpu.sync_copy(uoy_h.at[pl.ds(base, ROW)], uoy)
        pltpu.sync_copy(uoz_h.at[pl.ds(base, ROW)], uoz)
        pltpu.sync_copy(mp_h.at[pl.ds(base, ROW)], mp)
        pltpu.sync_copy(mo_h.at[pl.ds(base, ROW)], mo)
        pltpu.sync_copy(thp_h, thp)

        onehot = lane == seg
        th_all = _T_MIN * jnp.exp(thp[...] * _LOG_RANGE)
        th = jnp.sum(jnp.where(onehot, th_all, jnp.float32(0.0)))

        def body1(i, carry):
            accp, acch = carry
            sl = pl.ds(i * LANES, LANES)
            dux = upx[sl] - uox[sl]
            duy = upy[sl] - uoy[sl]
            duz = upz[sl] - uoz[sl]
            s2 = dux * dux + duy * duy + duz * duz
            close = s2 < th
            vv = jnp.where(close, s2 / th, jnp.float32(0.0))
            dm = mp[sl] - mo[sl]
            zz = dm / _SIGMA
            pmag = _COEF * jnp.exp(np.float32(-0.5) * zz * zz)
            cf = jnp.where(close, jnp.float32(1.0), jnp.float32(0.0))
            p = jnp.where(close, (jnp.float32(1.0) - vv) * pmag, jnp.float32(0.0))
            p_buf[sl] = p
            c_buf[sl] = cf
            return accp + p, acch + cf

        zero16 = jnp.zeros((LANES,), jnp.float32)
        accp, acch = lax.fori_loop(0, NITER, body1, (zero16, zero16))
        ps = jnp.sum(accp)
        hs = jnp.sum(acch)
        den = jnp.maximum(ps, jnp.float32(1e-30))

        def body2(i, accl):
            sl = pl.ds(i * LANES, LANES)
            t = jnp.maximum(p_buf[sl] / den, jnp.float32(1e-30))
            return accl + c_buf[sl] * _logf(t)

        accl = lax.fori_loop(0, NITER, body2, zero16)
        ls = jnp.sum(accl)

        stage_p[...] = jnp.where(onehot, ps, jnp.float32(0.0))
        stage_hh[...] = jnp.where(onehot, hs, jnp.float32(0.0))
        stage_l[...] = jnp.where(onehot, ls, jnp.float32(0.0))
        pltpu.sync_copy(stage_p, score_h.at[seg])
        pltpu.sync_copy(stage_hh, hits_h.at[seg])
        pltpu.sync_copy(stage_l, ll_h.at[seg])


def kernel(u_pred, mag_pred, u_obs, mag_obs, thresh_s2_param):
    upx, upy, upz = u_pred[:, 0], u_pred[:, 1], u_pred[:, 2]
    uox, uoy, uoz = u_obs[:, 0], u_obs[:, 1], u_obs[:, 2]
    score2, hits2, ll2 = _tscore(
        upx, upy, upz, uox, uoy, uoz, mag_pred, mag_obs, thresh_s2_param
    )
    diag = jnp.arange(N_SEG)
    join = lambda o: o[diag, diag]
    return join(score2), join(hits2), join(ll2)
